# baseline (speedup 1.0000x reference)
import jax
import jax.numpy as jnp
from jax.experimental import pallas as pl
from jax.experimental.pallas import tpu as pltpu


def _ln(x, g, b):
    mu = jnp.mean(x, axis=-1, keepdims=True)
    var = jnp.var(x, axis=-1, keepdims=True)
    return (x - mu) * jax.lax.rsqrt(var + 1e-5) * g + b


def _bn(x, g, b):
    return x * jax.lax.rsqrt(jnp.float32(1.0 + 1e-5)) * g + b


def _gcn(x, src, dst, W, bias):
    h = x @ W
    deg = jnp.zeros((x.shape[0],), dtype=h.dtype).at[dst].add(1.0)
    dinv = jax.lax.rsqrt(jnp.maximum(deg, 1e-12))
    norm = dinv[src] * dinv[dst]
    out = jnp.zeros_like(h).at[dst].add(h[src] * norm[:, None])
    return out + bias


def _cls_body(h_ref, w1_ref, b1_ref, g1_ref, be1_ref, w2_ref, b2_ref, g2_ref,
              be2_ref, w3_ref, b3_ref, out_ref):
    h = h_ref[...]
    h = jax.nn.relu(_ln(h @ w1_ref[...] + b1_ref[...], g1_ref[...], be1_ref[...]))
    h = jax.nn.relu(_ln(h @ w2_ref[...] + b2_ref[...], g2_ref[...], be2_ref[...]))
    out_ref[...] = h @ w3_ref[...] + b3_ref[...]


def kernel(x, edge_index, ln_g, ln_b, W1, b1, g1, be1, W2, b2, g2, be2, W3, b3,
           g3, be3, W4, b4, g4, be4, Wc1, bc1, lg1, lb1, Wc2, bc2, lg2, lb2,
           Wc3, bc3):
    n = x.shape[0]
    loop = jnp.arange(n, dtype=edge_index.dtype)
    src = jnp.concatenate([edge_index[0], loop])
    dst = jnp.concatenate([edge_index[1], loop])
    h = _ln(x, ln_g, ln_b)
    h = jax.nn.relu(_bn(_gcn(h, src, dst, W1, b1), g1, be1))
    h = jax.nn.relu(_bn(_gcn(h, src, dst, W2, b2), g2, be2))
    h = jax.nn.relu(_bn(_gcn(h, src, dst, W3, b3), g3, be3))
    h = jax.nn.relu(_bn(_gcn(h, src, dst, W4, b4), g4, be4))
    out = pl.pallas_call(
        _cls_body,
        out_shape=jax.ShapeDtypeStruct((n, Wc3.shape[1]), jnp.float32),
    )(h, Wc1, bc1, lg1, lb1, Wc2, bc2, lg2, lb2, Wc3, bc3)
    return out


# speedup vs baseline: 8.8615x; 8.8615x over previous
import functools

import jax
import jax.numpy as jnp
from jax import lax
from jax.experimental import pallas as pl
from jax.experimental.pallas import tpu as pltpu
from jax.experimental.pallas import tpu_sc as plsc

N = 10000
E = 320000
NT = 32
R = 320
NPAD = NT * R
SCAN = 3200
NCH = E // SCAN
CAP = E + 4352
GCH = 128

_mesh = plsc.VectorSubcoreMesh(core_axis_name="c", subcore_axis_name="s")
_sc_params = pltpu.CompilerParams(needs_layout_passes=False)


def _tile_id():
    return lax.axis_index("c") * 16 + lax.axis_index("s")


def _bin_body(src_hbm, dst_hbm, bin_src, bin_ldst, counts_hbm, deg_hbm,
              sbuf, dbuf, csrc, cldst, deg_acc, cvec):
    t = _tile_id()
    base = t * R
    zero16f = jnp.zeros((16,), jnp.float32)
    ones16f = jnp.ones((16,), jnp.float32)
    dummy_src = lax.iota(jnp.int32, 16) * 64
    dummy_ldst = jnp.full((16,), R, jnp.int32)

    for i in range(R // 16):
        deg_acc[pl.ds(i * 16, 16)] = zero16f

    def chunk_body(c, off):
        pltpu.sync_copy(src_hbm.at[pl.ds(c * SCAN, SCAN)], sbuf)
        pltpu.sync_copy(dst_hbm.at[pl.ds(c * SCAN, SCAN)], dbuf)

        def vreg_body(j, cnt):
            d = dbuf[pl.ds(j * 16, 16)]
            s = sbuf[pl.ds(j * 16, 16)]
            ld = d - base
            m = (ld >= 0) & (ld < R)
            ldc = jnp.where(m, ld, R)
            plsc.addupdate_scatter(deg_acc, [ldc], ones16f, mask=m)
            plsc.store_compressed(csrc.at[pl.ds(cnt, 16)], s, mask=m)
            plsc.store_compressed(cldst.at[pl.ds(cnt, 16)], ld, mask=m)
            return cnt + jnp.sum(jnp.where(m, 1, 0))

        cnt = lax.fori_loop(0, SCAN // 16, vreg_body, 0)
        off = pl.multiple_of(off, 8)
        csrc[pl.ds(cnt, 16)] = dummy_src
        cldst[pl.ds(cnt, 16)] = dummy_ldst
        cnt_pad = (cnt + 7) & ~7
        pltpu.sync_copy(csrc, bin_src.at[pl.ds(t * CAP + off, SCAN + 16)])
        pltpu.sync_copy(cldst, bin_ldst.at[pl.ds(t * CAP + off, SCAN + 16)])
        return off + cnt_pad

    off = pl.multiple_of(lax.fori_loop(0, NCH, chunk_body, 0, unroll=False), 8)

    for k in range(GCH // 16):
        csrc[pl.ds(k * 16, 16)] = dummy_src
        cldst[pl.ds(k * 16, 16)] = dummy_ldst
    pltpu.sync_copy(csrc.at[pl.ds(0, GCH)], bin_src.at[pl.ds(t * CAP + off, GCH)])
    pltpu.sync_copy(cldst.at[pl.ds(0, GCH)], bin_ldst.at[pl.ds(t * CAP + off, GCH)])
    total = ((off + GCH - 1) // GCH) * GCH

    cvec[...] = jnp.full((16,), 0, jnp.int32) + total
    pltpu.sync_copy(cvec, counts_hbm.at[pl.ds(t * 16, 16)])
    pltpu.sync_copy(deg_acc, deg_hbm.at[pl.ds(t * R, R)])


@functools.partial(jax.jit, static_argnums=())
def _bin(src, dst):
    f = pl.kernel(
        _bin_body,
        out_type=(
            jax.ShapeDtypeStruct((NT * CAP,), jnp.int32),
            jax.ShapeDtypeStruct((NT * CAP,), jnp.int32),
            jax.ShapeDtypeStruct((NT * 16,), jnp.int32),
            jax.ShapeDtypeStruct((NPAD,), jnp.float32),
        ),
        mesh=_mesh,
        compiler_params=_sc_params,
        scratch_types=[
            pltpu.VMEM((SCAN,), jnp.int32),
            pltpu.VMEM((SCAN,), jnp.int32),
            pltpu.VMEM((SCAN + 16,), jnp.int32),
            pltpu.VMEM((SCAN + 16,), jnp.int32),
            pltpu.VMEM((R,), jnp.float32),
            pltpu.VMEM((16,), jnp.int32),
        ],
    )
    return f(src, dst)


RT = R + 1


def _seg_body(d, p_hbm, bin_src, bin_ldst, counts_hbm, acc_hbm,
              src_v, ldst_v, rows_v, zbuf, acc_sh, cnt_v, sem):
    t = _tile_id()
    s = lax.axis_index("s")
    rbase = s * RT

    def zrow(r, carry):
        for k in range(d // 16):
            zbuf[r, pl.ds(k * 16, 16)] = jnp.zeros((16,), jnp.float32)
        return carry

    lax.fori_loop(0, RT, zrow, 0, unroll=False)
    pltpu.sync_copy(zbuf, acc_sh.at[pl.ds(rbase, RT)])

    pltpu.sync_copy(counts_hbm.at[pl.ds(t * 16, 16)], cnt_v)
    nch = jnp.max(cnt_v[...]) // GCH

    def chunk(c, carry):
        o = pl.multiple_of(t * CAP + c * GCH, 8)
        pltpu.sync_copy(bin_src.at[pl.ds(o, GCH)], src_v)
        pltpu.sync_copy(bin_ldst.at[pl.ds(o, GCH)], ldst_v.at[0])
        for k in range(GCH // 16):
            ldst_v[0, pl.ds(k * 16, 16)] = (
                ldst_v[0, pl.ds(k * 16, 16)] + rbase
            )
        pltpu.async_copy(p_hbm.at[src_v], rows_v, sem).wait()
        pltpu.sync_copy(rows_v, acc_sh.at[ldst_v.at[0]], add=True)
        return carry

    lax.fori_loop(0, nch, chunk, 0, unroll=False)
    pltpu.sync_copy(acc_sh.at[pl.ds(rbase, R)], acc_hbm.at[pl.ds(t * R, R)])


def _seg(p, bin_src, bin_ldst, counts, d):
    f = pl.kernel(
        functools.partial(_seg_body, d),
        out_type=jax.ShapeDtypeStruct((NPAD, d), jnp.float32),
        mesh=_mesh,
        compiler_params=_sc_params,
        scratch_types=[
            pltpu.VMEM((GCH,), jnp.int32),
            pltpu.VMEM((1, GCH), jnp.int32),
            pltpu.VMEM((GCH, d), jnp.float32),
            pltpu.VMEM((RT, d), jnp.float32),
            pltpu.VMEM_SHARED((16 * RT, d), jnp.float32),
            pltpu.VMEM((16,), jnp.int32),
            pltpu.SemaphoreType.DMA,
        ],
    )
    return f(p, bin_src, bin_ldst, counts)


BR = 1024


def _ln_rows(h, g, b):
    mu = jnp.mean(h, axis=-1, keepdims=True)
    var = jnp.var(h, axis=-1, keepdims=True)
    return (h - mu) * lax.rsqrt(var + 1e-5) * g + b


_BN_SC = 0.9999950000374997


def _tca_body(x_ref, deg_ref, lng_ref, lnb_ref, w_ref, p_ref):
    dinv = lax.rsqrt(deg_ref[...] + 1.0)
    h = _ln_rows(x_ref[...], lng_ref[...], lnb_ref[...])
    p_ref[...] = (h @ w_ref[...]) * dinv


def _tca(xp, deg, ln_g, ln_b, W1):
    grid = (NPAD // BR,)
    return pl.pallas_call(
        _tca_body,
        grid=grid,
        in_specs=[
            pl.BlockSpec((BR, 128), lambda i: (i, 0)),
            pl.BlockSpec((BR, 1), lambda i: (i, 0)),
            pl.BlockSpec((1, 128), lambda i: (0, 0)),
            pl.BlockSpec((1, 128), lambda i: (0, 0)),
            pl.BlockSpec((128, 128), lambda i: (0, 0)),
        ],
        out_specs=pl.BlockSpec((BR, 128), lambda i: (i, 0)),
        out_shape=jax.ShapeDtypeStruct((NPAD, 128), jnp.float32),
    )(xp, deg, ln_g.reshape(1, -1), ln_b.reshape(1, -1), W1)


def _tcb_body(acc_ref, p_ref, deg_ref, b_ref, g_ref, be_ref, w_ref, out_ref):
    dinv = lax.rsqrt(deg_ref[...] + 1.0)
    pre = (acc_ref[...] + p_ref[...]) * dinv + b_ref[...]
    h = jax.nn.relu(pre * (_BN_SC * g_ref[...]) + be_ref[...])
    out_ref[...] = (h @ w_ref[...]) * dinv


def _tcb(acc, p, deg, b, g, be, W, din, dout):
    grid = (NPAD // BR,)
    return pl.pallas_call(
        _tcb_body,
        grid=grid,
        in_specs=[
            pl.BlockSpec((BR, din), lambda i: (i, 0)),
            pl.BlockSpec((BR, din), lambda i: (i, 0)),
            pl.BlockSpec((BR, 1), lambda i: (i, 0)),
            pl.BlockSpec((1, din), lambda i: (0, 0)),
            pl.BlockSpec((1, din), lambda i: (0, 0)),
            pl.BlockSpec((1, din), lambda i: (0, 0)),
            pl.BlockSpec((din, dout), lambda i: (0, 0)),
        ],
        out_specs=pl.BlockSpec((BR, dout), lambda i: (i, 0)),
        out_shape=jax.ShapeDtypeStruct((NPAD, dout), jnp.float32),
    )(acc, p, deg, b.reshape(1, -1), g.reshape(1, -1), be.reshape(1, -1), W)


def _tce_body(acc_ref, p_ref, deg_ref, b_ref, g_ref, be_ref,
              wc1_ref, bc1_ref, lg1_ref, lb1_ref,
              wc2_ref, bc2_ref, lg2_ref, lb2_ref,
              wc3_ref, bc3_ref, out_ref):
    dinv = lax.rsqrt(deg_ref[...] + 1.0)
    pre = (acc_ref[...] + p_ref[...]) * dinv + b_ref[...]
    h = jax.nn.relu(pre * (_BN_SC * g_ref[...]) + be_ref[...])[:, :32]
    h = jax.nn.relu(_ln_rows(h @ wc1_ref[...] + bc1_ref[...],
                             lg1_ref[...], lb1_ref[...]))
    h = jax.nn.relu(_ln_rows(h @ wc2_ref[...] + bc2_ref[...],
                             lg2_ref[...], lb2_ref[...]))
    out_ref[...] = h @ wc3_ref[...] + bc3_ref[...]


def _tce(acc, p, deg, b4, g4, be4, Wc1, bc1, lg1, lb1, Wc2, bc2, lg2, lb2,
         Wc3, bc3):
    grid = (NPAD // BR,)
    row = lambda v: v.reshape(1, -1)
    full = lambda a, b: pl.BlockSpec((a, b), lambda i: (0, 0))
    return pl.pallas_call(
        _tce_body,
        grid=grid,
        in_specs=[
            pl.BlockSpec((BR, 128), lambda i: (i, 0)),
            pl.BlockSpec((BR, 128), lambda i: (i, 0)),
            pl.BlockSpec((BR, 1), lambda i: (i, 0)),
            full(1, 128), full(1, 128), full(1, 128),
            full(32, 16), full(1, 16), full(1, 16), full(1, 16),
            full(16, 8), full(1, 8), full(1, 8), full(1, 8),
            full(8, 8), full(1, 8),
        ],
        out_specs=pl.BlockSpec((BR, 8), lambda i: (i, 0)),
        out_shape=jax.ShapeDtypeStruct((NPAD, 8), jnp.float32),
    )(acc, p, deg, row(b4), row(g4), row(be4),
      Wc1, row(bc1), row(lg1), row(lb1),
      Wc2, row(bc2), row(lg2), row(lb2),
      Wc3, row(bc3))


def kernel(x, edge_index, ln_g, ln_b, W1, b1, g1, be1, W2, b2, g2, be2, W3, b3,
           g3, be3, W4, b4, g4, be4, Wc1, bc1, lg1, lb1, Wc2, bc2, lg2, lb2,
           Wc3, bc3):
    src = edge_index[0].astype(jnp.int32)
    dst = edge_index[1].astype(jnp.int32)
    bin_src, bin_ldst, counts, degc = _bin(src, dst)
    deg = degc.reshape(NPAD, 1)
    xp = jnp.pad(x, ((0, NPAD - N), (0, 0)))

    W3p = jnp.pad(W3, ((0, 0), (0, 64)))
    b3p = jnp.pad(b3, (0, 64))
    g3p = jnp.pad(g3, (0, 64))
    be3p = jnp.pad(be3, (0, 64))
    W4p = jnp.pad(W4, ((0, 64), (0, 96)))
    b4p = jnp.pad(b4, (0, 96))
    g4p = jnp.pad(g4, (0, 96))
    be4p = jnp.pad(be4, (0, 96))

    p1 = _tca(xp, deg, ln_g, ln_b, W1)
    a1 = _seg(p1, bin_src, bin_ldst, counts, 128)
    p2 = _tcb(a1, p1, deg, b1, g1, be1, W2, 128, 128)
    a2 = _seg(p2, bin_src, bin_ldst, counts, 128)
    p3 = _tcb(a2, p2, deg, b2, g2, be2, W3p, 128, 128)
    a3 = _seg(p3, bin_src, bin_ldst, counts, 128)
    p4 = _tcb(a3, p3, deg, b3p, g3p, be3p, W4p, 128, 128)
    a4 = _seg(p4, bin_src, bin_ldst, counts, 128)
    out = _tce(a4, p4, deg, b4p, g4p, be4p, Wc1, bc1, lg1, lb1,
               Wc2, bc2, lg2, lb2, Wc3, bc3)
    return out[:N]
